# Initial kernel scaffold; baseline (speedup 1.0000x reference)
import jax
import jax.numpy as jnp
from jax.experimental import pallas as pl


def kernel(data, segment_ids):
    raise NotImplementedError("write your pallas kernel here")


# speedup vs baseline: 7.4211x; 7.4211x over previous
import functools

import jax
import jax.numpy as jnp
from jax import lax
from jax.experimental import pallas as pl
from jax.experimental.pallas import tpu as pltpu
from jax.experimental.pallas import tpu_sc as plsc

NUM_TYPES = 100
N_SAMPLES = 3_200_000
N_PROPS = 16

NW = 32
ROWS_PER_W = N_SAMPLES // NW
CHUNK = 2000
NCHUNKS = ROWS_PER_W // CHUNK
UNROLL = 8


def _sc_partials(data, ids):
  mesh = plsc.VectorSubcoreMesh(core_axis_name="c", subcore_axis_name="s")

  @functools.partial(
      pl.kernel,
      out_type=[
          jax.ShapeDtypeStruct((NW, NUM_TYPES * N_PROPS), jnp.float32),
          jax.ShapeDtypeStruct((NW, NUM_TYPES * N_PROPS), jnp.float32),
      ],
      mesh=mesh,
      compiler_params=pltpu.CompilerParams(
          needs_layout_passes=False, use_tc_tiling_on_sc=False),
      scratch_types=[
          pltpu.VMEM((CHUNK, N_PROPS), jnp.float32),
          pltpu.VMEM((CHUNK, N_PROPS), jnp.float32),
          pltpu.VMEM((CHUNK,), jnp.int32),
          pltpu.VMEM((CHUNK,), jnp.int32),
          pltpu.VMEM((NUM_TYPES * N_PROPS,), jnp.float32),
          pltpu.VMEM((NUM_TYPES * N_PROPS,), jnp.float32),
          pltpu.SemaphoreType.DMA,
          pltpu.SemaphoreType.DMA,
          pltpu.SemaphoreType.DMA,
          pltpu.SemaphoreType.DMA,
      ],
  )
  def k(data_hbm, ids_hbm, y2_hbm, cnt_hbm, buf0, buf1, idsb0, idsb1,
        acc, cnt, sd0, sd1, si0, si1):
    wid = lax.axis_index("c") * 16 + lax.axis_index("s")
    row0 = wid * ROWS_PER_W
    iota16 = lax.iota(jnp.int32, 16)
    zeros16 = jnp.zeros((16,), jnp.float32)
    ones16 = jnp.ones((16,), jnp.float32)

    def zbody(kk, _):
      acc[pl.ds(kk * 16, 16)] = zeros16
      cnt[pl.ds(kk * 16, 16)] = zeros16
      return 0
    lax.fori_loop(0, NUM_TYPES, zbody, 0)

    def start(n, buf, idsb, sd, si):
      base = row0 + n * CHUNK
      pltpu.make_async_copy(data_hbm.at[pl.ds(base, CHUNK)], buf, sd).start()
      pltpu.make_async_copy(ids_hbm.at[pl.ds(base, CHUNK)], idsb, si).start()

    def wait(n, buf, idsb, sd, si):
      base = row0 + n * CHUNK
      pltpu.make_async_copy(data_hbm.at[pl.ds(base, CHUNK)], buf, sd).wait()
      pltpu.make_async_copy(ids_hbm.at[pl.ds(base, CHUNK)], idsb, si).wait()

    def process(buf, idsb):
      first = idsb[pl.ds(0, 16)][0]
      last = idsb[pl.ds(CHUNK - 16, 16)][15]
      uniform = first == last

      @pl.when(uniform)
      def _fast():
        def body(j, accs):
          out = []
          for u in range(UNROLL):
            v = buf[j * UNROLL + u, :]
            out.append(accs[u] + v * v)
          return tuple(out)
        accs = lax.fori_loop(0, CHUNK // UNROLL, body,
                             tuple(zeros16 for _ in range(UNROLL)))
        tot = accs[0]
        for u in range(1, UNROLL):
          tot = tot + accs[u]
        idx = jnp.full((16,), first * 16, jnp.int32) + iota16
        plsc.addupdate_scatter(acc, [idx], tot)
        plsc.addupdate_scatter(cnt, [idx],
                               jnp.full((16,), float(CHUNK), jnp.float32))

      @pl.when(jnp.logical_not(uniform))
      def _slow():
        def body(g, _):
          segs = idsb[pl.ds(g * 16, 16)]
          for u in range(16):
            idx = jnp.full((16,), segs[u] * 16, jnp.int32) + iota16
            v = buf[g * 16 + u, :]
            plsc.addupdate_scatter(acc, [idx], v * v)
            plsc.addupdate_scatter(cnt, [idx], ones16)
          return 0
        lax.fori_loop(0, CHUNK // 16, body, 0)

    start(0, buf0, idsb0, sd0, si0)
    start(1, buf1, idsb1, sd1, si1)

    def outer(kk, _):
      n0 = 2 * kk
      wait(n0, buf0, idsb0, sd0, si0)
      process(buf0, idsb0)
      start(n0 + 2, buf0, idsb0, sd0, si0)
      wait(n0 + 1, buf1, idsb1, sd1, si1)
      process(buf1, idsb1)
      start(n0 + 3, buf1, idsb1, sd1, si1)
      return 0
    lax.fori_loop(0, NCHUNKS // 2 - 1, outer, 0)

    wait(NCHUNKS - 2, buf0, idsb0, sd0, si0)
    process(buf0, idsb0)
    wait(NCHUNKS - 1, buf1, idsb1, sd1, si1)
    process(buf1, idsb1)

    pltpu.sync_copy(acc, y2_hbm.at[wid])
    pltpu.sync_copy(cnt, cnt_hbm.at[wid])

  return k(data, ids)


def _tc_finalize(y2p, cntp):
  def body(y2_ref, cnt_ref, o_ref):
    y2 = jnp.sum(y2_ref[...], axis=0)
    c = jnp.sum(cnt_ref[...], axis=0)
    o_ref[...] = jnp.where(c > 0.0, jnp.sqrt(y2 / jnp.maximum(c, 1.0)),
                           jnp.float32(1.0))

  return pl.pallas_call(
      body,
      out_shape=jax.ShapeDtypeStruct((NUM_TYPES * N_PROPS,), jnp.float32),
  )(y2p, cntp)


@jax.jit
def kernel(data, segment_ids):
  ids = segment_ids.astype(jnp.int32)
  y2p, cntp = _sc_partials(data, ids)
  return _tc_finalize(y2p, cntp).reshape(NUM_TYPES, N_PROPS)


# speedup vs baseline: 70.1171x; 9.4483x over previous
import functools

import jax
import jax.numpy as jnp
from jax import lax
from jax.experimental import pallas as pl
from jax.experimental.pallas import tpu as pltpu
from jax.experimental.pallas import tpu_sc as plsc

NUM_TYPES = 100
N_SAMPLES = 3_200_000
N_PROPS = 16

NW = 32
NBLK = 25
CHUNK = NBLK * 128
NCHUNKS = N_SAMPLES // CHUNK
SLOTS = -(-NCHUNKS // NW)


def _sc_partials(data4, ids):
  mesh = plsc.VectorSubcoreMesh(core_axis_name="c", subcore_axis_name="s")

  @functools.partial(
      pl.kernel,
      out_type=[
          jax.ShapeDtypeStruct((NW, NUM_TYPES * N_PROPS), jnp.float32),
          jax.ShapeDtypeStruct((NW, NUM_TYPES * N_PROPS), jnp.float32),
      ],
      mesh=mesh,
      compiler_params=pltpu.CompilerParams(
          needs_layout_passes=False, use_tc_tiling_on_sc=False),
      scratch_types=[
          pltpu.VMEM((NBLK, 8, 128), jnp.float32),
          pltpu.VMEM((NBLK, 8, 128), jnp.float32),
          pltpu.VMEM((NBLK, 8, 128), jnp.float32),
          pltpu.VMEM((NBLK, 8, 128), jnp.float32),
          pltpu.VMEM((CHUNK,), jnp.int32),
          pltpu.VMEM((CHUNK,), jnp.int32),
          pltpu.VMEM((NUM_TYPES * N_PROPS,), jnp.float32),
          pltpu.VMEM((NUM_TYPES * N_PROPS,), jnp.float32),
          pltpu.VMEM((256,), jnp.float32),
          pltpu.SemaphoreType.DMA,
          pltpu.SemaphoreType.DMA,
          pltpu.SemaphoreType.DMA,
          pltpu.SemaphoreType.DMA,
      ],
  )
  def k(data_hbm, ids_hbm, y2_hbm, cnt_hbm, lo0, hi0, lo1, hi1, idsb0, idsb1,
        acc, cnt, tt, sd0, sd1, si0, si1):
    wid = lax.axis_index("c") * 16 + lax.axis_index("s")
    nc = jnp.where(wid < NCHUNKS - (SLOTS - 1) * NW, SLOTS, SLOTS - 1)
    iota16 = lax.iota(jnp.int32, 16)
    iota16x16 = iota16 * 16
    zeros16 = jnp.zeros((16,), jnp.float32)
    ones16 = jnp.ones((16,), jnp.float32)

    def zbody(kk, _):
      acc[pl.ds(kk * 16, 16)] = zeros16
      cnt[pl.ds(kk * 16, 16)] = zeros16
      return 0
    lax.fori_loop(0, NUM_TYPES, zbody, 0)

    def chunk_of(slot):
      return wid + jnp.minimum(slot, nc - 1) * NW

    def start(slot, lo, hi, idsb, sd, si):
      c = chunk_of(slot)
      b = c * NBLK
      pltpu.make_async_copy(data_hbm.at[0, pl.ds(b, NBLK)], lo, sd).start()
      pltpu.make_async_copy(data_hbm.at[1, pl.ds(b, NBLK)], hi, sd).start()
      pltpu.make_async_copy(ids_hbm.at[pl.ds(c * CHUNK, CHUNK)], idsb,
                            si).start()

    def wait(slot, lo, hi, idsb, sd, si):
      c = chunk_of(slot)
      b = c * NBLK
      pltpu.make_async_copy(data_hbm.at[0, pl.ds(b, NBLK)], lo, sd).wait()
      pltpu.make_async_copy(data_hbm.at[1, pl.ds(b, NBLK)], hi, sd).wait()
      pltpu.make_async_copy(ids_hbm.at[pl.ds(c * CHUNK, CHUNK)], idsb,
                            si).wait()

    def lanesum_from_tt():
      tot = plsc.load_gather(tt, [iota16x16])
      for l in range(1, 16):
        tot = tot + plsc.load_gather(tt, [iota16x16 + l])
      return tot

    def flush_accp(accp, seg, n_samples):
      for p in range(16):
        tt[pl.ds(p * 16, 16)] = accp[p]
      tot = lanesum_from_tt()
      idx = jnp.full((16,), seg * 16, jnp.int32) + iota16
      plsc.addupdate_scatter(acc, [idx], tot)
      plsc.addupdate_scatter(cnt, [idx],
                             jnp.full((16,), n_samples, jnp.float32))

    def accum_block(lo, hi, blk, accp):
      out = list(accp)
      for half, buf in ((0, lo), (1, hi)):
        for j in range(8):
          p = half * 8 + j
          a = out[p]
          for kk in range(8):
            v = buf[blk, j, pl.ds(kk * 16, 16)]
            a = a + v * v
          out[p] = a
      return tuple(out)

    def process(lo, hi, idsb):
      first = idsb[pl.ds(0, 16)][0]
      last = idsb[pl.ds(CHUNK - 16, 16)][15]
      uniform = first == last

      @pl.when(uniform)
      def _fast():
        accp = lax.fori_loop(
            0, NBLK, lambda blk, accs: accum_block(lo, hi, blk, accs),
            tuple(zeros16 for _ in range(16)))
        flush_accp(accp, first, float(CHUNK))

      @pl.when(jnp.logical_not(uniform))
      def _slow():
        def blk_body(blk, _):
          boff = blk * 128
          bfirst = idsb[pl.ds(boff, 16)][0]
          blast = idsb[pl.ds(boff + 112, 16)][15]

          @pl.when(bfirst == blast)
          def _ublock():
            accp = accum_block(lo, hi, blk, tuple(zeros16 for _ in range(16)))
            flush_accp(accp, bfirst, 128.0)

          @pl.when(jnp.logical_not(bfirst == blast))
          def _bblock():
            for kk in range(8):
              segs = idsb[pl.ds(boff + kk * 16, 16)]
              for half, buf in ((0, lo), (1, hi)):
                for j in range(8):
                  tt[pl.ds((half * 8 + j) * 16, 16)] = (
                      buf[blk, j, pl.ds(kk * 16, 16)])
              for l in range(16):
                col = plsc.load_gather(tt, [iota16x16 + l])
                idx = jnp.full((16,), segs[l] * 16, jnp.int32) + iota16
                plsc.addupdate_scatter(acc, [idx], col * col)
                plsc.addupdate_scatter(cnt, [idx], ones16)
          return 0
        lax.fori_loop(0, NBLK, blk_body, 0)

    start(0, lo0, hi0, idsb0, sd0, si0)
    start(1, lo1, hi1, idsb1, sd1, si1)

    def outer(kk, _):
      n0 = 2 * kk
      wait(n0, lo0, hi0, idsb0, sd0, si0)

      @pl.when(n0 < nc)
      def _p0():
        process(lo0, hi0, idsb0)
      start(n0 + 2, lo0, hi0, idsb0, sd0, si0)

      wait(n0 + 1, lo1, hi1, idsb1, sd1, si1)

      @pl.when(n0 + 1 < nc)
      def _p1():
        process(lo1, hi1, idsb1)
      start(n0 + 3, lo1, hi1, idsb1, sd1, si1)
      return 0
    lax.fori_loop(0, SLOTS // 2, outer, 0)

    wait(SLOTS, lo0, hi0, idsb0, sd0, si0)
    wait(SLOTS + 1, lo1, hi1, idsb1, sd1, si1)

    pltpu.sync_copy(acc, y2_hbm.at[wid])
    pltpu.sync_copy(cnt, cnt_hbm.at[wid])

  return k(data4, ids)


def _tc_finalize(y2p, cntp):
  def body(y2_ref, cnt_ref, o_ref):
    y2 = jnp.sum(y2_ref[...], axis=0)
    c = jnp.sum(cnt_ref[...], axis=0)
    o_ref[...] = jnp.where(c > 0.0, jnp.sqrt(y2 / jnp.maximum(c, 1.0)),
                           jnp.float32(1.0))

  return pl.pallas_call(
      body,
      out_shape=jax.ShapeDtypeStruct((NUM_TYPES * N_PROPS,), jnp.float32),
  )(y2p, cntp)


@jax.jit
def kernel(data, segment_ids):
  ids = segment_ids.astype(jnp.int32)
  data4 = data.T.reshape(2, 8, N_SAMPLES // 128, 128).transpose(0, 2, 1, 3)
  y2p, cntp = _sc_partials(data4, ids)
  return _tc_finalize(y2p, cntp).reshape(NUM_TYPES, N_PROPS)


# speedup vs baseline: 71.8216x; 1.0243x over previous
import functools

import jax
import jax.numpy as jnp
from jax import lax
from jax.experimental import pallas as pl
from jax.experimental.pallas import tpu as pltpu
from jax.experimental.pallas import tpu_sc as plsc

NUM_TYPES = 100
N_SAMPLES = 3_200_000
N_PROPS = 16

NW = 32
NBLK = 25
CHUNK = NBLK * 128
NCHUNKS = N_SAMPLES // CHUNK
SLOTS = -(-NCHUNKS // NW)


def _sc_partials(data4, ids):
  mesh = plsc.VectorSubcoreMesh(core_axis_name="c", subcore_axis_name="s")

  @functools.partial(
      pl.kernel,
      out_type=jax.ShapeDtypeStruct((NW * 2 * NUM_TYPES * N_PROPS,),
                                    jnp.float32),
      mesh=mesh,
      compiler_params=pltpu.CompilerParams(
          needs_layout_passes=False, use_tc_tiling_on_sc=False),
      scratch_types=[
          pltpu.VMEM((NBLK, 8, 128), jnp.float32),
          pltpu.VMEM((NBLK, 8, 128), jnp.float32),
          pltpu.VMEM((NBLK, 8, 128), jnp.float32),
          pltpu.VMEM((NBLK, 8, 128), jnp.float32),
          pltpu.VMEM((CHUNK,), jnp.int32),
          pltpu.VMEM((CHUNK,), jnp.int32),
          pltpu.VMEM((NUM_TYPES * N_PROPS,), jnp.float32),
          pltpu.VMEM((NUM_TYPES * N_PROPS,), jnp.float32),
          pltpu.VMEM((256,), jnp.float32),
          pltpu.SemaphoreType.DMA,
          pltpu.SemaphoreType.DMA,
          pltpu.SemaphoreType.DMA,
          pltpu.SemaphoreType.DMA,
      ],
  )
  def k(data_hbm, ids_hbm, out_hbm, lo0, hi0, lo1, hi1, idsb0, idsb1,
        acc, cnt, tt, sd0, sd1, si0, si1):
    wid = lax.axis_index("c") * 16 + lax.axis_index("s")
    nc = jnp.where(wid < NCHUNKS - (SLOTS - 1) * NW, SLOTS, SLOTS - 1)
    iota16 = lax.iota(jnp.int32, 16)
    iota16x16 = iota16 * 16
    zeros16 = jnp.zeros((16,), jnp.float32)
    ones16 = jnp.ones((16,), jnp.float32)

    def zbody(kk, _):
      acc[pl.ds(kk * 16, 16)] = zeros16
      cnt[pl.ds(kk * 16, 16)] = zeros16
      return 0
    lax.fori_loop(0, NUM_TYPES, zbody, 0)

    def chunk_of(slot):
      return wid + jnp.minimum(slot, nc - 1) * NW

    def start(slot, lo, hi, idsb, sd, si):
      c = chunk_of(slot)
      b = c * NBLK
      pltpu.make_async_copy(data_hbm.at[0, pl.ds(b, NBLK)], lo, sd).start()
      pltpu.make_async_copy(data_hbm.at[1, pl.ds(b, NBLK)], hi, sd).start()
      pltpu.make_async_copy(ids_hbm.at[pl.ds(c * CHUNK, CHUNK)], idsb,
                            si).start()

    def wait(slot, lo, hi, idsb, sd, si):
      c = chunk_of(slot)
      b = c * NBLK
      pltpu.make_async_copy(data_hbm.at[0, pl.ds(b, NBLK)], lo, sd).wait()
      pltpu.make_async_copy(data_hbm.at[1, pl.ds(b, NBLK)], hi, sd).wait()
      pltpu.make_async_copy(ids_hbm.at[pl.ds(c * CHUNK, CHUNK)], idsb,
                            si).wait()

    def lanesum_from_tt():
      tot = plsc.load_gather(tt, [iota16x16])
      for l in range(1, 16):
        tot = tot + plsc.load_gather(tt, [iota16x16 + l])
      return tot

    def flush_accp(accp, seg, n_samples):
      for p in range(16):
        tt[pl.ds(p * 16, 16)] = accp[p]
      tot = lanesum_from_tt()
      idx = jnp.full((16,), seg * 16, jnp.int32) + iota16
      plsc.addupdate_scatter(acc, [idx], tot)
      plsc.addupdate_scatter(cnt, [idx],
                             jnp.full((16,), n_samples, jnp.float32))

    def accum_block(lo, hi, blk, accp):
      out = list(accp)
      for half, buf in ((0, lo), (1, hi)):
        for j in range(8):
          p = half * 8 + j
          a = out[p]
          for kk in range(8):
            v = buf[blk, j, pl.ds(kk * 16, 16)]
            a = a + v * v
          out[p] = a
      return tuple(out)

    def process(lo, hi, idsb):
      first = idsb[pl.ds(0, 16)][0]
      last = idsb[pl.ds(CHUNK - 16, 16)][15]
      uniform = first == last

      @pl.when(uniform)
      def _fast():
        accp = lax.fori_loop(
            0, NBLK, lambda blk, accs: accum_block(lo, hi, blk, accs),
            tuple(zeros16 for _ in range(16)))
        flush_accp(accp, first, float(CHUNK))

      @pl.when(jnp.logical_not(uniform))
      def _slow():
        def blk_body(blk, _):
          boff = blk * 128
          bfirst = idsb[pl.ds(boff, 16)][0]
          blast = idsb[pl.ds(boff + 112, 16)][15]

          @pl.when(bfirst == blast)
          def _ublock():
            accp = accum_block(lo, hi, blk, tuple(zeros16 for _ in range(16)))
            flush_accp(accp, bfirst, 128.0)

          @pl.when(jnp.logical_not(bfirst == blast))
          def _bblock():
            for kk in range(8):
              segs = idsb[pl.ds(boff + kk * 16, 16)]
              for half, buf in ((0, lo), (1, hi)):
                for j in range(8):
                  tt[pl.ds((half * 8 + j) * 16, 16)] = (
                      buf[blk, j, pl.ds(kk * 16, 16)])
              for l in range(16):
                col = plsc.load_gather(tt, [iota16x16 + l])
                idx = jnp.full((16,), segs[l] * 16, jnp.int32) + iota16
                plsc.addupdate_scatter(acc, [idx], col * col)
                plsc.addupdate_scatter(cnt, [idx], ones16)
          return 0
        lax.fori_loop(0, NBLK, blk_body, 0)

    start(0, lo0, hi0, idsb0, sd0, si0)
    start(1, lo1, hi1, idsb1, sd1, si1)

    def outer(kk, _):
      n0 = 2 * kk
      wait(n0, lo0, hi0, idsb0, sd0, si0)

      @pl.when(n0 < nc)
      def _p0():
        process(lo0, hi0, idsb0)
      start(n0 + 2, lo0, hi0, idsb0, sd0, si0)

      wait(n0 + 1, lo1, hi1, idsb1, sd1, si1)

      @pl.when(n0 + 1 < nc)
      def _p1():
        process(lo1, hi1, idsb1)
      start(n0 + 3, lo1, hi1, idsb1, sd1, si1)
      return 0
    lax.fori_loop(0, SLOTS // 2, outer, 0)

    wait(SLOTS, lo0, hi0, idsb0, sd0, si0)
    wait(SLOTS + 1, lo1, hi1, idsb1, sd1, si1)

    base = wid * 2 * NUM_TYPES * N_PROPS
    pltpu.sync_copy(acc, out_hbm.at[pl.ds(base, NUM_TYPES * N_PROPS)])
    pltpu.sync_copy(
        cnt, out_hbm.at[pl.ds(base + NUM_TYPES * N_PROPS,
                              NUM_TYPES * N_PROPS)])

  return k(data4, ids)


def _tc_finalize(parts):
  d = NUM_TYPES * N_PROPS

  def body(p_ref, o_ref):
    y2 = p_ref[pl.ds(0, d)]
    c = p_ref[pl.ds(d, d)]
    for w in range(1, NW):
      y2 = y2 + p_ref[pl.ds(w * 2 * d, d)]
      c = c + p_ref[pl.ds(w * 2 * d + d, d)]
    o_ref[...] = jnp.where(c > 0.0, jnp.sqrt(y2 / jnp.maximum(c, 1.0)),
                           jnp.float32(1.0))

  return pl.pallas_call(
      body,
      out_shape=jax.ShapeDtypeStruct((d,), jnp.float32),
  )(parts)


@jax.jit
def kernel(data, segment_ids):
  ids = segment_ids.astype(jnp.int32)
  data4 = data.T.reshape(2, 8, N_SAMPLES // 128, 128).transpose(0, 2, 1, 3)
  parts = _sc_partials(data4, ids)
  return _tc_finalize(parts).reshape(NUM_TYPES, N_PROPS)
